# baseline (speedup 1.0000x reference)
import functools

import jax
import jax.numpy as jnp
from jax import lax
from jax.experimental import pallas as pl
from jax.experimental.pallas import tpu as pltpu
from jax.experimental.pallas import tpu_sc as plsc

F = 128
KP = 64
KD = 32
NDCM = 3
CUT = 10.0
NITER = 2


def _edge_basis(d, K, cutoff):
    centers = jnp.linspace(0.0, cutoff, K)
    gamma = 4.0
    rbf = jnp.exp(-gamma * (d[:, None] - centers[None, :]) ** 2)
    fc = jnp.where(d < cutoff, 0.5 * (jnp.cos(jnp.pi * d / cutoff) + 1.0), 0.0)
    return rbf * fc[:, None]


_NC = 2
_NS = 16
_CH = 80
_N_ATOMS = 10000
_N_EDGES = 320000
_EPS = _N_EDGES // _NS
_NCHUNK = _EPS // _CH
_ROWN = 640
_ZR = _CH


def _sc_agg_body(h_hbm, g_hbm, ep_hbm, ed_hbm, srcr_hbm, dstr_hbm, out_hbm,
                 src_b, dst_b, rows_v, e_v, agg_sh, gsem, esem, isem):
    c = lax.axis_index("c")
    s = lax.axis_index("s")

    ntile = jnp.where(s == _NS - 1, 400 // _ZR, _ROWN // _ZR)

    def zrow(i, carry):
        for jj in range(F // 16):
            rows_v[0, i, pl.ds(jj * 16, 16)] = jnp.zeros((16,), jnp.float32)
        return carry

    lax.fori_loop(0, _CH, zrow, 0)

    def ztile(t, carry):
        r0 = pl.multiple_of(s * _ROWN + t * _ZR, 8)
        pltpu.sync_copy(rows_v.at[0], agg_sh.at[pl.ds(r0, _ZR)])
        return carry

    lax.fori_loop(0, ntile, ztile, 0)
    plsc.subcore_barrier()

    def idx_copies(k, j):
        return (pltpu.make_async_copy(srcr_hbm.at[s, k], src_b.at[j], isem),
                pltpu.make_async_copy(dstr_hbm.at[s, k], dst_b.at[j], isem))

    def gather_copy(j, b, tbl):
        return pltpu.make_async_copy(tbl.at[src_b.at[j]], rows_v.at[b], gsem)

    def gather_start(j, b):
        @pl.when(c == 0)
        def _():
            gather_copy(j, b, h_hbm).start()

        @pl.when(c == 1)
        def _():
            gather_copy(j, b, g_hbm).start()

    def gather_wait(j, b):
        @pl.when(c == 0)
        def _():
            gather_copy(j, b, h_hbm).wait()

        @pl.when(c == 1)
        def _():
            gather_copy(j, b, g_hbm).wait()

    def e_copy(k, tbl):
        start = pl.multiple_of(s * _EPS + k * _CH, 8)
        return pltpu.make_async_copy(tbl.at[pl.ds(start, _CH)], e_v, esem)

    def e_start(k):
        @pl.when(c == 0)
        def _():
            e_copy(k, ep_hbm).start()

        @pl.when(c == 1)
        def _():
            e_copy(k, ed_hbm).start()

    def e_wait(k):
        @pl.when(c == 0)
        def _():
            e_copy(k, ep_hbm).wait()

        @pl.when(c == 1)
        def _():
            e_copy(k, ed_hbm).wait()

    for j in range(3):
        a, bcp = idx_copies(j, j)
        a.start()
        bcp.start()
    a, bcp = idx_copies(0, 0)
    a.wait()
    bcp.wait()
    gather_start(0, 0)
    e_start(0)

    def chunk_body(k, b):
        nb = 1 - b

        @pl.when(k + 1 < _NCHUNK)
        def _():
            j1 = lax.rem(k + 1, 4)
            a1, b1 = idx_copies(k + 1, j1)
            a1.wait()
            b1.wait()
            gather_start(j1, nb)

        gather_wait(lax.rem(k, 4), b)
        e_wait(k)

        for i in range(_CH):
            for jj in range(F // 16):
                sl = pl.ds(jj * 16, 16)
                rows_v[b, i, sl] = rows_v[b, i, sl] * e_v[i, sl]

        @pl.when(k + 1 < _NCHUNK)
        def _():
            e_start(k + 1)

        pltpu.sync_copy(rows_v.at[b], agg_sh.at[dst_b.at[lax.rem(k, 4)]], add=True)

        @pl.when(k + 3 < _NCHUNK)
        def _():
            j3 = lax.rem(k + 3, 4)
            a3, b3 = idx_copies(k + 3, j3)
            a3.start()
            b3.start()

    def pair(t, carry):
        chunk_body(2 * t, 0)
        chunk_body(2 * t + 1, 1)
        return carry

    lax.fori_loop(0, _NCHUNK // 2, pair, 0)
    plsc.subcore_barrier()

    def otile(t, carry):
        r0 = pl.multiple_of(s * _ROWN + t * _ZR, 8)
        pltpu.sync_copy(agg_sh.at[pl.ds(r0, _ZR)], out_hbm.at[c, pl.ds(r0, _ZR)])
        return carry

    lax.fori_loop(0, ntile, otile, 0)


def _sc_agg2(h, g, e_pn, e_d, src_r, dst_r):
    k = pl.kernel(
        _sc_agg_body,
        out_type=jax.ShapeDtypeStruct((_NC, _N_ATOMS, F), jnp.float32),
        mesh=plsc.VectorSubcoreMesh(core_axis_name="c", subcore_axis_name="s"),
        scratch_types=[
            pltpu.VMEM((4, _CH), jnp.int32),
            pltpu.VMEM((4, _CH), jnp.int32),
            pltpu.VMEM((2, _CH, F), jnp.float32),
            pltpu.VMEM((_CH, F), jnp.float32),
            pltpu.VMEM_SHARED((_N_ATOMS, F), jnp.float32),
            pltpu.SemaphoreType.DMA,
            pltpu.SemaphoreType.DMA,
            pltpu.SemaphoreType.DMA,
        ],
    )
    return k(h, g, e_pn, e_d, src_r, dst_r)


def _mlp_update_body(h_ref, agg_ref, w1_ref, b1_ref, w2_ref, b2_ref, out_ref):
    h = h_ref[...]
    agg = agg_ref[...]
    x = jnp.tanh((h + agg) @ w1_ref[...] + b1_ref[...])
    out_ref[...] = x + jnp.tanh(x @ w2_ref[...] + b2_ref[...])


def _mlp_update(h, agg, w1, b1, w2, b2):
    N = h.shape[0]
    RB = 2000
    grid = N // RB
    return pl.pallas_call(
        _mlp_update_body,
        grid=(grid,),
        in_specs=[
            pl.BlockSpec((RB, F), lambda i: (i, 0)),
            pl.BlockSpec((RB, F), lambda i: (i, 0)),
            pl.BlockSpec((F, F), lambda i: (0, 0)),
            pl.BlockSpec((1, F), lambda i: (0, 0)),
            pl.BlockSpec((F, F), lambda i: (0, 0)),
            pl.BlockSpec((1, F), lambda i: (0, 0)),
        ],
        out_specs=pl.BlockSpec((RB, F), lambda i: (i, 0)),
        out_shape=jax.ShapeDtypeStruct((N, F), jnp.float32),
    )(h, agg, w1, b1.reshape(1, F), w2, b2.reshape(1, F))


def kernel(atomic_numbers, positions, dst_idx, src_idx, batch_segments, batch_size, batch_mask, atom_mask, embed_pn, rbfW_pn, W1_pn, b1_pn, W2_pn, b2_pn, Wq, bq, We, be, Wf, bf, embed_dcm, rbfW_dcm, W1_dcm, b1_dcm, W2_dcm, b2_dcm, Wmono, bmono, Wdipo, bdipo):
    N = atomic_numbers.shape[0]
    num_segments_static = batch_mask.shape[0]
    rij = positions[dst_idx] - positions[src_idx]
    d = jnp.sqrt(jnp.sum(rij * rij, axis=-1) + 1e-12)
    src_r = src_idx.reshape(_NS, _NCHUNK, _CH).astype(jnp.int32)
    dst_r = dst_idx.reshape(_NS, _NCHUNK, _CH).astype(jnp.int32)

    e_pn = _edge_basis(d, KP, CUT) @ rbfW_pn
    e_d = _edge_basis(d, KD, CUT) @ rbfW_dcm
    h = embed_pn[atomic_numbers]
    g = embed_dcm[atomic_numbers]
    for _ in range(NITER):
        agg2 = _sc_agg2(h, g, e_pn, e_d, src_r, dst_r)
        h = _mlp_update(h, agg2[0], W1_pn, b1_pn, W2_pn, b2_pn)
        g = _mlp_update(g, agg2[1], W1_dcm, b1_dcm, W2_dcm, b2_dcm)
    charges = h @ Wq + bq
    charges_sq = jnp.squeeze(charges)
    charges_masked = charges_sq * atom_mask
    bs_zero = (jnp.asarray(batch_size) * 0).astype(charges.dtype)
    sum_charges = jax.ops.segment_sum(charges_masked, segment_ids=batch_segments, num_segments=num_segments_static) + bs_zero
    energy_atom = jnp.squeeze(h @ We + be) * atom_mask
    energy = jax.ops.segment_sum(energy_atom, segment_ids=batch_segments, num_segments=num_segments_static) * batch_mask
    forces = (h @ Wf + bf) * atom_mask[:, None]
    dipoles = jax.ops.segment_sum(charges_masked[:, None] * positions, segment_ids=batch_segments, num_segments=num_segments_static)

    mono_dist = g @ Wmono + bmono
    dipo_dist = positions[:, :, None] + (g @ Wdipo + bdipo).reshape(N, 3, NDCM)

    return {
        'energy': energy,
        'forces': forces,
        'dipoles': dipoles,
        'charges': charges,
        'sum_charges': sum_charges,
        'mono_dist': mono_dist,
        'dipo_dist': dipo_dist,
        'charges_as_mono': charges_sq,
        'coulomb_energy': jnp.array(0.0),
        'coulomb_lambda': jnp.array(0.0),
    }


# speedup vs baseline: 1.7055x; 1.7055x over previous
import functools

import jax
import jax.numpy as jnp
from jax import lax
from jax.experimental import pallas as pl
from jax.experimental.pallas import tpu as pltpu
from jax.experimental.pallas import tpu_sc as plsc

F = 128
KP = 64
KD = 32
NDCM = 3
CUT = 10.0
NITER = 2


def _edge_basis(d, K, cutoff):
    centers = jnp.linspace(0.0, cutoff, K)
    gamma = 4.0
    rbf = jnp.exp(-gamma * (d[:, None] - centers[None, :]) ** 2)
    fc = jnp.where(d < cutoff, 0.5 * (jnp.cos(jnp.pi * d / cutoff) + 1.0), 0.0)
    return rbf * fc[:, None]


_NC = 2
_NS = 16
_CH = 40
_N_ATOMS = 10000
_N_EDGES = 320000
_EPW = _N_EDGES // (_NC * _NS)
_NCHUNK = _EPW // _CH
_GCH = 80
_GNCHUNK = _EPW // _GCH
_ROWN = 640
_ZR = _CH


def _sc_agg_body(h_hbm, e_hbm, srcr_hbm, dstr_hbm, out_hbm,
                 src_b, dst_b, rows_v, e_v, agg_sh, gsem, esem, isem):
    c = lax.axis_index("c")
    s = lax.axis_index("s")
    wid = c * _NS + s

    ntile = jnp.where(s == _NS - 1, 400 // _ZR, _ROWN // _ZR)

    def zrow(i, carry):
        for jj in range(F // 16):
            rows_v[0, i, pl.ds(jj * 16, 16)] = jnp.zeros((16,), jnp.float32)
        return carry

    lax.fori_loop(0, _CH, zrow, 0)

    def ztile(t, carry):
        r0 = pl.multiple_of(s * _ROWN + t * _ZR, 8)
        pltpu.sync_copy(rows_v.at[0], agg_sh.at[pl.ds(r0, _ZR)])
        return carry

    lax.fori_loop(0, ntile, ztile, 0)
    plsc.subcore_barrier()

    def idx_copies(k, j):
        return (pltpu.make_async_copy(srcr_hbm.at[wid, k], src_b.at[j], isem),
                pltpu.make_async_copy(dstr_hbm.at[wid, k], dst_b.at[j], isem))

    def gather_copy2(j, b):
        return pltpu.make_async_copy(h_hbm.at[src_b.at[j]], rows_v.at[b], gsem)

    def e_copy(k, b):
        start = pl.multiple_of(wid * _EPW + k * _CH, 8)
        return pltpu.make_async_copy(e_hbm.at[pl.ds(start, _CH)], e_v.at[b], esem)

    for j in range(3):
        a, bcp = idx_copies(j, j)
        a.start()
        bcp.start()
    a, bcp = idx_copies(0, 0)
    a.wait()
    bcp.wait()
    gather_copy2(0, 0).start()
    e_copy(0, 0).start()

    def chunk_body(k, b):
        nb = 1 - b

        @pl.when(k + 1 < _NCHUNK)
        def _():
            j1 = lax.rem(k + 1, 4)
            a1, b1 = idx_copies(k + 1, j1)
            a1.wait()
            b1.wait()
            gather_copy2(j1, nb).start()
            e_copy(k + 1, nb).start()

        gather_copy2(lax.rem(k, 4), b).wait()
        e_copy(k, b).wait()

        for i in range(_CH):
            for jj in range(F // 16):
                sl = pl.ds(jj * 16, 16)
                rows_v[b, i, sl] = rows_v[b, i, sl] * e_v[b, i, sl]
        pltpu.sync_copy(rows_v.at[b], agg_sh.at[dst_b.at[lax.rem(k, 4)]], add=True)

        @pl.when(k + 3 < _NCHUNK)
        def _():
            j3 = lax.rem(k + 3, 4)
            a3, b3 = idx_copies(k + 3, j3)
            a3.start()
            b3.start()

    def pair(t, carry):
        chunk_body(2 * t, 0)
        chunk_body(2 * t + 1, 1)
        return carry

    lax.fori_loop(0, _NCHUNK // 2, pair, 0)
    plsc.subcore_barrier()

    def otile(t, carry):
        r0 = pl.multiple_of(s * _ROWN + t * _ZR, 8)
        pltpu.sync_copy(agg_sh.at[pl.ds(r0, _ZR)], out_hbm.at[c, pl.ds(r0, _ZR)])
        return carry

    lax.fori_loop(0, ntile, otile, 0)


def _sc_agg(h, e, src_r, dst_r):
    k = pl.kernel(
        _sc_agg_body,
        out_type=jax.ShapeDtypeStruct((_NC, _N_ATOMS, F), jnp.float32),
        mesh=plsc.VectorSubcoreMesh(core_axis_name="c", subcore_axis_name="s"),
        scratch_types=[
            pltpu.VMEM((4, _CH), jnp.int32),
            pltpu.VMEM((4, _CH), jnp.int32),
            pltpu.VMEM((2, _CH, F), jnp.float32),
            pltpu.VMEM((2, _CH, F), jnp.float32),
            pltpu.VMEM_SHARED((_N_ATOMS, F), jnp.float32),
            pltpu.SemaphoreType.DMA,
            pltpu.SemaphoreType.DMA,
            pltpu.SemaphoreType.DMA,
        ],
    )
    parts = k(h, e, src_r, dst_r)
    return parts[0] + parts[1]


def _sc_geom_body(pos_hbm, srcg_hbm, dstg_hbm, out_hbm,
                  src_b, dst_b, pd_v, ps_v, gsem, isem):
    c = lax.axis_index("c")
    s = lax.axis_index("s")
    wid = c * _NS + s

    def idx_copies(k, j):
        return (pltpu.make_async_copy(srcg_hbm.at[wid, k], src_b.at[j], isem),
                pltpu.make_async_copy(dstg_hbm.at[wid, k], dst_b.at[j], isem))

    def gpair(j, b):
        return (pltpu.make_async_copy(pos_hbm.at[dst_b.at[j]], pd_v.at[b], gsem),
                pltpu.make_async_copy(pos_hbm.at[src_b.at[j]], ps_v.at[b], gsem))

    for j in range(3):
        a, bcp = idx_copies(j, j)
        a.start()
        bcp.start()
    a, bcp = idx_copies(0, 0)
    a.wait()
    bcp.wait()
    g1, g2 = gpair(0, 0)
    g1.start()
    g2.start()

    def chunk(k, carry):
        b = lax.rem(k, 2)
        nb = 1 - b

        @pl.when(k + 1 < _GNCHUNK)
        def _():
            j1 = lax.rem(k + 1, 4)
            a1, b1 = idx_copies(k + 1, j1)
            a1.wait()
            b1.wait()
            n1, n2 = gpair(j1, nb)
            n1.start()
            n2.start()

        w1, w2 = gpair(lax.rem(k, 4), b)
        w1.wait()
        w2.wait()
        base = pl.multiple_of(wid * _EPW + k * _GCH, 8)
        pltpu.sync_copy(pd_v.at[b], out_hbm.at[0, pl.ds(base, _GCH)])
        pltpu.sync_copy(ps_v.at[b], out_hbm.at[1, pl.ds(base, _GCH)])

        @pl.when(k + 3 < _GNCHUNK)
        def _():
            j3 = lax.rem(k + 3, 4)
            a3, b3 = idx_copies(k + 3, j3)
            a3.start()
            b3.start()

        return carry

    lax.fori_loop(0, _GNCHUNK, chunk, 0)


def _sc_geom(pos8, srcg, dstg):
    k = pl.kernel(
        _sc_geom_body,
        out_type=jax.ShapeDtypeStruct((2, _N_EDGES, 8), jnp.float32),
        mesh=plsc.VectorSubcoreMesh(core_axis_name="c", subcore_axis_name="s"),
        compiler_params=pltpu.CompilerParams(use_tc_tiling_on_sc=False),
        scratch_types=[
            pltpu.VMEM((4, _GCH), jnp.int32),
            pltpu.VMEM((4, _GCH), jnp.int32),
            pltpu.VMEM((2, _GCH, 8), jnp.float32),
            pltpu.VMEM((2, _GCH, 8), jnp.float32),
            pltpu.SemaphoreType.DMA,
            pltpu.SemaphoreType.DMA,
        ],
    )
    return k(pos8, srcg, dstg)


def _mlp_update_body(h_ref, agg_ref, w1_ref, b1_ref, w2_ref, b2_ref, out_ref):
    h = h_ref[...]
    agg = agg_ref[...]
    x = jnp.tanh((h + agg) @ w1_ref[...] + b1_ref[...])
    out_ref[...] = x + jnp.tanh(x @ w2_ref[...] + b2_ref[...])


def _mlp_update(h, agg, w1, b1, w2, b2):
    N = h.shape[0]
    RB = 2000
    grid = N // RB
    return pl.pallas_call(
        _mlp_update_body,
        grid=(grid,),
        in_specs=[
            pl.BlockSpec((RB, F), lambda i: (i, 0)),
            pl.BlockSpec((RB, F), lambda i: (i, 0)),
            pl.BlockSpec((F, F), lambda i: (0, 0)),
            pl.BlockSpec((1, F), lambda i: (0, 0)),
            pl.BlockSpec((F, F), lambda i: (0, 0)),
            pl.BlockSpec((1, F), lambda i: (0, 0)),
        ],
        out_specs=pl.BlockSpec((RB, F), lambda i: (i, 0)),
        out_shape=jax.ShapeDtypeStruct((N, F), jnp.float32),
    )(h, agg, w1, b1.reshape(1, F), w2, b2.reshape(1, F))


def kernel(atomic_numbers, positions, dst_idx, src_idx, batch_segments, batch_size, batch_mask, atom_mask, embed_pn, rbfW_pn, W1_pn, b1_pn, W2_pn, b2_pn, Wq, bq, We, be, Wf, bf, embed_dcm, rbfW_dcm, W1_dcm, b1_dcm, W2_dcm, b2_dcm, Wmono, bmono, Wdipo, bdipo):
    N = atomic_numbers.shape[0]
    num_segments_static = batch_mask.shape[0]
    src_i32 = src_idx.astype(jnp.int32)
    dst_i32 = dst_idx.astype(jnp.int32)
    src_r = src_i32.reshape(_NC * _NS, _NCHUNK, _CH)
    dst_r = dst_i32.reshape(_NC * _NS, _NCHUNK, _CH)
    srcg = src_i32.reshape(_NC * _NS, _GNCHUNK, _GCH)
    dstg = dst_i32.reshape(_NC * _NS, _GNCHUNK, _GCH)

    pos8 = jnp.pad(positions, ((0, 0), (0, 5)))
    geo = _sc_geom(pos8, srcg, dstg)
    rij = geo[0, :, :3] - geo[1, :, :3]
    d = jnp.sqrt(jnp.sum(rij * rij, axis=-1) + 1e-12)

    e_pn = _edge_basis(d, KP, CUT) @ rbfW_pn
    e_d = _edge_basis(d, KD, CUT) @ rbfW_dcm
    h = embed_pn[atomic_numbers]
    g = embed_dcm[atomic_numbers]
    for _ in range(NITER):
        aggP = _sc_agg(h, e_pn, src_r, dst_r)
        aggD = _sc_agg(g, e_d, src_r, dst_r)
        h = _mlp_update(h, aggP, W1_pn, b1_pn, W2_pn, b2_pn)
        g = _mlp_update(g, aggD, W1_dcm, b1_dcm, W2_dcm, b2_dcm)
    charges = h @ Wq + bq
    charges_sq = jnp.squeeze(charges)
    charges_masked = charges_sq * atom_mask
    bs_zero = (jnp.asarray(batch_size) * 0).astype(charges.dtype)
    sum_charges = jax.ops.segment_sum(charges_masked, segment_ids=batch_segments, num_segments=num_segments_static) + bs_zero
    energy_atom = jnp.squeeze(h @ We + be) * atom_mask
    energy = jax.ops.segment_sum(energy_atom, segment_ids=batch_segments, num_segments=num_segments_static) * batch_mask
    forces = (h @ Wf + bf) * atom_mask[:, None]
    dipoles = jax.ops.segment_sum(charges_masked[:, None] * positions, segment_ids=batch_segments, num_segments=num_segments_static)

    mono_dist = g @ Wmono + bmono
    dipo_dist = positions[:, :, None] + (g @ Wdipo + bdipo).reshape(N, 3, NDCM)

    return {
        'energy': energy,
        'forces': forces,
        'dipoles': dipoles,
        'charges': charges,
        'sum_charges': sum_charges,
        'mono_dist': mono_dist,
        'dipo_dist': dipo_dist,
        'charges_as_mono': charges_sq,
        'coulomb_energy': jnp.array(0.0),
        'coulomb_lambda': jnp.array(0.0),
    }
